# baseline (speedup 1.0000x reference)
import functools

import numpy as np
import jax
import jax.numpy as jnp
from jax import lax
from jax.experimental import pallas as pl
from jax.experimental.pallas import tpu as pltpu
from jax.experimental.pallas import tpu_sc as plsc

VOCAB = 100000
EMBED = 768
BATCH = 32
SEQ = 512
MAX_POS = 512

NC, NS, L = 2, 16, 16
NW = NC * NS
TOK_PER_W = (BATCH * SEQ) // NW
CH = 16
NCHUNK = TOK_PER_W // CH
NBUF = 4
NROUND = NCHUNK // NBUF


def _positional_encoding_np():
    pos = np.arange(MAX_POS)[:, np.newaxis]
    i = np.arange(EMBED)[np.newaxis, :]
    angle_rates = 1 / np.power(10000, 2 * (i // 2) / np.float32(EMBED))
    angle_rads = pos * angle_rates
    sines = np.sin(angle_rads[:, 0::2])
    cosines = np.cos(angle_rads[:, 1::2])
    return np.concatenate([sines, cosines], axis=-1).astype(np.float32)


_POS_ENC = _positional_encoding_np()


def _comb_body(pos_ref, seg_ref, out_ref):
    out_ref[...] = seg_ref[...][:, None, :] + pos_ref[...][None, :, :]


def _build_comb(seg_table):
    pos = jnp.asarray(_POS_ENC)
    out = pl.pallas_call(
        _comb_body,
        out_shape=jax.ShapeDtypeStruct((3, MAX_POS, EMBED), jnp.float32),
    )(pos, seg_table)
    return out.reshape(3 * MAX_POS, EMBED)


_MESH = plsc.VectorSubcoreMesh(
    core_axis_name="c", subcore_axis_name="s", num_cores=NC, num_subcores=NS)


@functools.partial(
    pl.kernel,
    out_type=jax.ShapeDtypeStruct((BATCH * SEQ, EMBED), jnp.float32),
    mesh=_MESH,
    scratch_types=[
        pltpu.VMEM((TOK_PER_W,), jnp.int32),
        pltpu.VMEM((TOK_PER_W,), jnp.int32),
        pltpu.VMEM((NBUF, CH, EMBED), jnp.float32),
        pltpu.VMEM((NBUF, CH, EMBED), jnp.float32),
        pltpu.SemaphoreType.DMA((NBUF,)),
        pltpu.SemaphoreType.DMA((NBUF,)),
        pltpu.SemaphoreType.DMA((NBUF,)),
    ],
)
def _embed_sc(tok_hbm, comb_hbm, x_hbm, seg_hbm, out_hbm,
              idx_v, idx2_v, rows_v, comb_v,
              gsem, csem, ssem):
    wid = lax.axis_index("s") * NC + lax.axis_index("c")
    base = wid * TOK_PER_W

    pltpu.sync_copy(x_hbm.at[wid], idx_v)
    pltpu.sync_copy(seg_hbm.at[wid], idx2_v)
    for i in range(TOK_PER_W // L):
        g = idx2_v[pl.ds(i * L, L)]
        idx2_v[pl.ds(i * L, L)] = g * MAX_POS + lax.iota(jnp.int32, L) + i * L

    def _descs(c, b):
        dt = pltpu.make_async_copy(
            tok_hbm.at[idx_v.at[pl.ds(c * CH, CH)]], rows_v.at[b], gsem.at[b])
        dp = pltpu.make_async_copy(
            comb_hbm.at[idx2_v.at[pl.ds(c * CH, CH)]], comb_v.at[b],
            csem.at[b])
        return dt, dp

    def kick(c, b):
        dt, dp = _descs(c, b)
        dt.start()
        dp.start()

    def _store_desc(c, b):
        return pltpu.make_async_copy(
            rows_v.at[b], out_hbm.at[pl.ds(base + c * CH, CH)], ssem.at[b])

    for b in range(NBUF):
        kick(b, b)

    def round_body(r, carry):
        for b in range(NBUF):
            c = r * NBUF + b
            dt, dp = _descs(c, b)
            dt.wait()
            dp.wait()

            def add_slice(k, acc, _b=b):
                for u in range(2):
                    sl = pl.ds((2 * k + u) * L, L)
                    for t in range(CH):
                        plsc.addupdate(
                            rows_v.at[_b, t, sl], comb_v[_b, t, sl])
                return acc

            lax.fori_loop(0, EMBED // (2 * L), add_slice, 0)
            _store_desc(c, b).start()
            bm = (b - 1) % NBUF
            cm = c - 1

            @pl.when(jnp.logical_and(cm >= 0, cm + NBUF < NCHUNK))
            def _(cm=cm, bm=bm):
                _store_desc(cm, bm).wait()
                kick(cm + NBUF, bm)
        return carry

    lax.fori_loop(0, NROUND, round_body, 0)
    for c in range(NCHUNK - NBUF, NCHUNK):
        _store_desc(c, c % NBUF).wait()


def kernel(x, pad_seg_embed_token, token_table, seg_table):
    comb = _build_comb(seg_table)
    out = _embed_sc(token_table, comb, x, pad_seg_embed_token)
    return out.reshape(BATCH, SEQ, EMBED)


# speedup vs baseline: 1.2674x; 1.2674x over previous
import functools

import numpy as np
import jax
import jax.numpy as jnp
from jax import lax
from jax.experimental import pallas as pl
from jax.experimental.pallas import tpu as pltpu
from jax.experimental.pallas import tpu_sc as plsc

VOCAB = 100000
EMBED = 768
BATCH = 32
SEQ = 512
MAX_POS = 512

NC, NS, L = 2, 16, 16
NW = NC * NS
TOK_PER_W = (BATCH * SEQ) // NW
CH = 16
NCHUNK = TOK_PER_W // CH
NBUF = 4
NROUND = NCHUNK // NBUF


def _positional_encoding_np():
    pos = np.arange(MAX_POS)[:, np.newaxis]
    i = np.arange(EMBED)[np.newaxis, :]
    angle_rates = 1 / np.power(10000, 2 * (i // 2) / np.float32(EMBED))
    angle_rads = pos * angle_rates
    sines = np.sin(angle_rads[:, 0::2])
    cosines = np.cos(angle_rads[:, 1::2])
    return np.concatenate([sines, cosines], axis=-1).astype(np.float32)


_POS_ENC = _positional_encoding_np()


def _comb_body(pos_ref, seg_ref, out_ref):
    out_ref[...] = seg_ref[...][:, None, :] + pos_ref[...][None, :, :]


def _build_comb(seg_table):
    pos = jnp.asarray(_POS_ENC)
    out = pl.pallas_call(
        _comb_body,
        out_shape=jax.ShapeDtypeStruct((3, MAX_POS, EMBED), jnp.float32),
    )(pos, seg_table)
    return out.reshape(3 * MAX_POS, EMBED)


_MESH = plsc.VectorSubcoreMesh(
    core_axis_name="c", subcore_axis_name="s", num_cores=NC, num_subcores=NS)


@functools.partial(
    pl.kernel,
    out_type=jax.ShapeDtypeStruct((BATCH * SEQ, EMBED), jnp.float32),
    mesh=_MESH,
    scratch_types=[
        pltpu.VMEM((TOK_PER_W,), jnp.int32),
        pltpu.VMEM((TOK_PER_W,), jnp.int32),
        pltpu.VMEM((NBUF, CH, EMBED), jnp.float32),
        pltpu.VMEM((NBUF, CH, EMBED), jnp.float32),
        pltpu.SemaphoreType.DMA((NBUF,)),
        pltpu.SemaphoreType.DMA((NBUF,)),
        pltpu.SemaphoreType.DMA((NBUF,)),
    ],
)
def _embed_sc(tok_hbm, comb_hbm, x_hbm, seg_hbm, out_hbm,
              idx_v, idx2_v, rows_v, comb_v,
              gsem, csem, ssem):
    wid = lax.axis_index("s") * NC + lax.axis_index("c")
    base = wid * TOK_PER_W

    pltpu.sync_copy(x_hbm.at[wid], idx_v)
    pltpu.sync_copy(seg_hbm.at[wid], idx2_v)
    for i in range(TOK_PER_W // L):
        g = idx2_v[pl.ds(i * L, L)]
        idx2_v[pl.ds(i * L, L)] = g * MAX_POS + lax.iota(jnp.int32, L) + i * L

    def _descs(c, b):
        dt = pltpu.make_async_copy(
            tok_hbm.at[idx_v.at[pl.ds(c * CH, CH)]], rows_v.at[b], gsem.at[b])
        dp = pltpu.make_async_copy(
            comb_hbm.at[idx2_v.at[pl.ds(c * CH, CH)]], comb_v.at[b],
            csem.at[b])
        return dt, dp

    def kick(c, b):
        dt, dp = _descs(c, b)
        dt.start()
        dp.start()

    def _store_desc(c, b):
        return pltpu.make_async_copy(
            rows_v.at[b], out_hbm.at[pl.ds(base + c * CH, CH)], ssem.at[b])

    for b in range(NBUF):
        kick(b, b)

    def round_body(r, carry):
        for b in range(NBUF):
            c = r * NBUF + b
            dt, dp = _descs(c, b)
            dt.wait()
            dp.wait()

            def add_slice(k, acc, _b=b):
                sl = pl.ds(k * L, L)
                for t in range(CH):
                    plsc.addupdate(
                        rows_v.at[_b, t, sl], comb_v[_b, t, sl])
                return acc

            lax.fori_loop(0, EMBED // L, add_slice, 0)
            _store_desc(c, b).start()
            bm = (b - 1) % NBUF
            cm = c - 1

            @pl.when(jnp.logical_and(cm >= 0, cm + NBUF < NCHUNK))
            def _(cm=cm, bm=bm):
                _store_desc(cm, bm).wait()
                kick(cm + NBUF, bm)
        return carry

    lax.fori_loop(0, NROUND, round_body, 0)
    for c in range(NCHUNK - NBUF, NCHUNK):
        _store_desc(c, c % NBUF).wait()


def kernel(x, pad_seg_embed_token, token_table, seg_table):
    comb = _build_comb(seg_table)
    out = _embed_sc(token_table, comb, x, pad_seg_embed_token)
    return out.reshape(BATCH, SEQ, EMBED)
